# Initial kernel scaffold; baseline (speedup 1.0000x reference)
import jax
import jax.numpy as jnp
from jax.experimental import pallas as pl


def kernel(x, mask_idx):
    raise NotImplementedError("write your pallas kernel here")


# speedup vs baseline: 1.4969x; 1.4969x over previous
import functools

import jax
import jax.numpy as jnp
from jax import lax
from jax.experimental import pallas as pl
from jax.experimental.pallas import tpu as pltpu
from jax.experimental.pallas import tpu_sc as plsc

B = 64
CHW = 3 * 512 * 512
G = 128
ROWS_PER_BATCH = 4608
TOTAL_ROWS = B * ROWS_PER_BATCH
NUM_WORKERS = 32
ROWS_PER_WORKER = TOTAL_ROWS // NUM_WORKERS
K = 128
NUM_CHUNKS = ROWS_PER_WORKER // K

_mesh = plsc.VectorSubcoreMesh(core_axis_name="c", subcore_axis_name="s")


@functools.partial(
    pl.kernel,
    mesh=_mesh,
    out_type=jax.ShapeDtypeStruct((TOTAL_ROWS, G), jnp.float32),
    scratch_types=[
        pltpu.VMEM((K,), jnp.int32),
        pltpu.VMEM((K, G), jnp.float32),
        pltpu.SemaphoreType.DMA,
    ],
)
def _masked_gather(table, rows, out, idx_v, buf_v, sem):
    wid = lax.axis_index("s") * 2 + lax.axis_index("c")
    base = wid * ROWS_PER_WORKER

    def body(i, carry):
        r0 = base + i * K
        pltpu.sync_copy(rows.at[pl.ds(r0, K)], idx_v)
        pltpu.async_copy(table.at[idx_v], buf_v, sem).wait()
        pltpu.sync_copy(buf_v, out.at[pl.ds(r0, K)])
        return carry

    lax.fori_loop(0, NUM_CHUNKS, body, 0)


def kernel(x, mask_idx):
    table = x.reshape(-1, G)
    gran = mask_idx[::G] >> 7
    rows = (jnp.arange(B, dtype=jnp.int32) * (CHW // G))[:, None] + gran[None, :]
    rows = rows.reshape(-1)
    out = _masked_gather(table, rows)
    return out.reshape(B, -1)
